# baseline (speedup 1.0000x reference)
import functools

import jax
import jax.numpy as jnp
from jax import lax
from jax.experimental import pallas as pl
from jax.experimental.pallas import tpu as pltpu

B, T, N, C = 4, 16, 512, 7
GEOM, AUX, K = 3, 4, 20
CONV = [32, 32, 32]
DM, NH, DH, FF, NC = 1024, 4, 256, 2048, 12


def _ln(x, g, b, eps=1e-5):
    mu = jnp.mean(x, -1, keepdims=True)
    v = jnp.mean((x - mu) ** 2, -1, keepdims=True)
    return (x - mu) / jnp.sqrt(v + eps) * g + b


def _sigmoid(x):
    return 1.0 / (1.0 + jnp.exp(-x))


def _dot(a, b):
    return jax.lax.dot_general(a, b, (((1,), (0,)), ((), ())),
                               preferred_element_type=jnp.float32)


def _dot_t(a, b):
    return jax.lax.dot_general(a, b, (((1,), (1,)), ((), ())),
                               preferred_element_type=jnp.float32)


def _edge_kernel(geom_ref, aux_ref,
                 eW1_0, eb1_0, eW2_0, eb2_0, aW1_0, ab1_0, aW2_0, ab2_0, lng_0, lnb_0,
                 eW1_1, eb1_1, eW2_1, eb2_1, aW1_1, ab1_1, aW2_1, ab2_1, lng_1, lnb_1,
                 eW1_2, eb1_2, eW2_2, eb2_2, aW1_2, ab1_2, aW2_2, ab2_2, lng_2, lnb_2,
                 lin1_W, lin1_b, out_ref, d2a_ref, d2b_ref, d2c_ref, d2d_ref,
                 cj_ref):
    layers = [
        (eW1_0, eb1_0, eW2_0, eb2_0, aW1_0, ab1_0, aW2_0, ab2_0, lng_0, lnb_0),
        (eW1_1, eb1_1, eW2_1, eb2_1, aW1_1, ab1_1, aW2_1, ab2_1, lng_1, lnb_1),
        (eW1_2, eb1_2, eW2_2, eb2_2, aW1_2, ab1_2, aW2_2, ab2_2, lng_2, lnb_2),
    ]
    x = geom_ref[0]
    aux = aux_ref[0]
    H = N // 4
    iota_h = lax.broadcasted_iota(jnp.int32, (H, N), 1)
    diag = lax.broadcasted_iota(jnp.int32, (N, N), 0) == \
        lax.broadcasted_iota(jnp.int32, (N, N), 1)
    xs = []
    for layer_refs in layers:
        (eW1, eb1, eW2, eb2, aW1, ab1, aW2, ab2, lng, lnb) = [
            r[...] for r in layer_refs]
        in_d = x.shape[-1]
        out_d = eW2.shape[-1]
        w = in_d + AUX
        Wa = eW1[:in_d, :]
        Wb = eW1[in_d:, :]
        P = _dot(x, Wa - Wb) + eb1[...]
        A1 = _dot(aux, aW1[:AUX, :]) + ab1[...]
        aWb = aW1[AUX:, :]
        cat = jnp.concatenate([x, aux], axis=1)

        sq = jnp.sum(x * x, axis=1, keepdims=True)
        d2 = sq + jnp.transpose(sq) - 2.0 * _dot_t(x, x)
        d2 = jnp.where(diag, 1e10, d2)
        d2a_ref[...] = d2[:H]
        d2b_ref[...] = d2[H:2 * H]
        d2c_ref[...] = d2[2 * H:3 * H]
        d2d_ref[...] = d2[3 * H:]

        def body(k, carry):
            for h, ref in ((0, d2a_ref), (1, d2b_ref), (2, d2c_ref),
                           (3, d2d_ref)):
                kc = ref[...]
                m = jnp.min(kc, axis=1, keepdims=True)
                idx = jnp.min(jnp.where(kc == m, iota_h, N), axis=1,
                              keepdims=True)
                oh = iota_h == idx
                catj = _dot(oh.astype(jnp.float32), cat)
                cj_ref[pl.ds(k * N + h * H, H), :w] = catj
                ref[...] = jnp.where(oh, jnp.float32(1e30), kc)
            return carry

        lax.fori_loop(0, K, body, 0)

        CJ = cj_ref[:, :w]
        z_a = jnp.zeros((in_d, 64), jnp.float32)
        z_b = jnp.zeros((AUX, out_d), jnp.float32)
        W1c = jnp.concatenate([
            jnp.concatenate([Wb, z_a], axis=1),
            jnp.concatenate([z_b, aWb], axis=1)], axis=0)
        z_c = jnp.zeros((out_d, 2 * out_d), jnp.float32)
        z_d = jnp.zeros((64, out_d), jnp.float32)
        W2c = jnp.concatenate([
            jnp.concatenate([eW2, z_c], axis=1),
            jnp.concatenate([z_d, aW2], axis=1)], axis=0)
        PA = jnp.concatenate([P, A1], axis=1)
        PAb = jnp.concatenate([PA] * K, axis=0)
        hg1 = jnp.maximum(PAb + _dot(CJ, W1c), 0.0)
        bc = jnp.concatenate([eb2, ab2], axis=1)
        hg2 = _dot(hg1, W2c) + bc
        h2 = jnp.maximum(hg2[:, :out_d], 0.0)
        gb = hg2[:, out_d:]
        mod = _sigmoid(gb[:, :out_d] + 1.0) * h2 + gb[:, out_d:]
        agg = jnp.max(mod.reshape(K, N, out_d), axis=0)
        x = jnp.maximum(_ln(agg, lng[...], lnb[...]), 0.0)
        xs.append(x)
    x_cat = jnp.concatenate(xs, axis=1)
    x_lin = jnp.maximum(_dot(x_cat, lin1_W[...]) + lin1_b[...], 0.0)
    out_ref[0, 0, :] = jnp.max(x_lin, axis=0)


def _tf_kernel(pooled_ref, pos_ref, Wq, bq, Wk, bk, Wv, bv, Wo, bo,
               ln1g, ln1b, W1, b1, W2, b2, ln2g, ln2b,
               hW0, hb0, hW1, hb1, hW2, hb2, hW3, hb3, out_ref):
    pos = pos_ref[...]
    seq = pooled_ref[...] + jnp.concatenate([pos] * B, axis=0)
    q = _dot(seq, Wq[...]) + bq[...]
    k = _dot(seq, Wk[...]) + bk[...]
    v = _dot(seq, Wv[...]) + bv[...]
    scale = 1.0 / jnp.sqrt(jnp.float32(DH))
    rows = []
    for b in range(B):
        cols = []
        for h in range(NH):
            qb = q[b * T:(b + 1) * T, h * DH:(h + 1) * DH]
            kb = k[b * T:(b + 1) * T, h * DH:(h + 1) * DH]
            vb = v[b * T:(b + 1) * T, h * DH:(h + 1) * DH]
            s = _dot_t(qb, kb) * scale
            s = s - jnp.max(s, axis=1, keepdims=True)
            e = jnp.exp(s)
            p = e / jnp.sum(e, axis=1, keepdims=True)
            cols.append(_dot(p, vb))
        rows.append(jnp.concatenate(cols, axis=1))
    ctx = jnp.concatenate(rows, axis=0)
    y = _ln(seq + _dot(ctx, Wo[...]) + bo[...], ln1g[...], ln1b[...])
    ff = _dot(jnp.maximum(_dot(y, W1[...]) + b1[...], 0.0), W2[...]) + b2[...]
    y = _ln(y + ff, ln2g[...], ln2b[...])
    feat = jnp.mean(y.reshape(B, T, DM), axis=1)
    h0 = jnp.maximum(_dot(feat, hW0[...]) + hb0[...], 0.0)
    h1 = jnp.maximum(_dot(h0, hW1[...]) + hb1[...], 0.0)
    h2 = jnp.maximum(_dot(h1, hW2[...]) + hb2[...], 0.0)
    out_ref[...] = _dot(h2, hW3[...]) + hb3[...]


def kernel(data, params):
    frames = B * T
    geom = data[..., :GEOM].reshape(frames, N, GEOM)
    aux = data[..., GEOM:GEOM + AUX].reshape(frames, N, AUX)

    edge_args = []
    edge_specs = []
    full = lambda s: pl.BlockSpec(s, lambda f: (0,) * len(s))
    for lp in params['edge']:
        for name in ('eW1', 'eb1', 'eW2', 'eb2', 'aW1', 'ab1', 'aW2', 'ab2',
                     'ln_g', 'ln_b'):
            w = lp[name]
            if w.ndim == 1:
                w = w.reshape(1, -1)
            edge_args.append(w)
            edge_specs.append(full(w.shape))

    lin1_W = params['lin1_W']
    lin1_b = params['lin1_b'].reshape(1, -1)

    pooled = pl.pallas_call(
        _edge_kernel,
        grid=(frames,),
        in_specs=[
            pl.BlockSpec((1, N, GEOM), lambda f: (f, 0, 0)),
            pl.BlockSpec((1, N, AUX), lambda f: (f, 0, 0)),
            *edge_specs,
            full(lin1_W.shape),
            full(lin1_b.shape),
        ],
        out_specs=pl.BlockSpec((1, 1, DM), lambda f: (f, 0, 0)),
        out_shape=jax.ShapeDtypeStruct((frames, 1, DM), jnp.float32),
        scratch_shapes=[pltpu.VMEM((N // 4, N), jnp.float32),
                        pltpu.VMEM((N // 4, N), jnp.float32),
                        pltpu.VMEM((N // 4, N), jnp.float32),
                        pltpu.VMEM((N // 4, N), jnp.float32),
                        pltpu.VMEM((K * N, CONV[0] + AUX), jnp.float32)],
    )(geom, aux, *edge_args, lin1_W, lin1_b)
    pooled = pooled.reshape(frames, DM)

    tf = params['tf']
    head = params['head']
    pos = params['pos'][0, :T, :]
    tf_args = [tf['Wq'], tf['bq'].reshape(1, -1), tf['Wk'], tf['bk'].reshape(1, -1),
               tf['Wv'], tf['bv'].reshape(1, -1), tf['Wo'], tf['bo'].reshape(1, -1),
               tf['ln1_g'].reshape(1, -1), tf['ln1_b'].reshape(1, -1),
               tf['W1'], tf['b1'].reshape(1, -1), tf['W2'], tf['b2'].reshape(1, -1),
               tf['ln2_g'].reshape(1, -1), tf['ln2_b'].reshape(1, -1),
               head[0]['W'], head[0]['b'].reshape(1, -1),
               head[1]['W'], head[1]['b'].reshape(1, -1),
               head[2]['W'], head[2]['b'].reshape(1, -1),
               head[3]['W'], head[3]['b'].reshape(1, -1)]

    out = pl.pallas_call(
        _tf_kernel,
        out_shape=jax.ShapeDtypeStruct((B, NC), jnp.float32),
    )(pooled, pos, *tf_args)
    return out


# speedup vs baseline: 1.1549x; 1.1549x over previous
import functools

import jax
import jax.numpy as jnp
from jax import lax
from jax.experimental import pallas as pl
from jax.experimental.pallas import tpu as pltpu

B, T, N, C = 4, 16, 512, 7
GEOM, AUX, K = 3, 4, 20
CONV = [32, 32, 32]
DM, NH, DH, FF, NC = 1024, 4, 256, 2048, 12


def _ln(x, g, b, eps=1e-5):
    mu = jnp.mean(x, -1, keepdims=True)
    v = jnp.mean((x - mu) ** 2, -1, keepdims=True)
    return (x - mu) / jnp.sqrt(v + eps) * g + b


def _sigmoid(x):
    return 1.0 / (1.0 + jnp.exp(-x))


def _dot(a, b):
    return jax.lax.dot_general(a, b, (((1,), (0,)), ((), ())),
                               preferred_element_type=jnp.float32)


def _dot_t(a, b):
    return jax.lax.dot_general(a, b, (((1,), (1,)), ((), ())),
                               preferred_element_type=jnp.float32)


def _edge_kernel(geom_ref, aux_ref,
                 eW1_0, eb1_0, eW2_0, eb2_0, aW1_0, ab1_0, aW2_0, ab2_0, lng_0, lnb_0,
                 eW1_1, eb1_1, eW2_1, eb2_1, aW1_1, ab1_1, aW2_1, ab2_1, lng_1, lnb_1,
                 eW1_2, eb1_2, eW2_2, eb2_2, aW1_2, ab1_2, aW2_2, ab2_2, lng_2, lnb_2,
                 lin1_W, lin1_b, out_ref, d2a_ref, d2b_ref, cj_ref):
    layers = [
        (eW1_0, eb1_0, eW2_0, eb2_0, aW1_0, ab1_0, aW2_0, ab2_0, lng_0, lnb_0),
        (eW1_1, eb1_1, eW2_1, eb2_1, aW1_1, ab1_1, aW2_1, ab2_1, lng_1, lnb_1),
        (eW1_2, eb1_2, eW2_2, eb2_2, aW1_2, ab1_2, aW2_2, ab2_2, lng_2, lnb_2),
    ]
    x = geom_ref[0]
    aux = aux_ref[0]
    H = N // 2
    iota_h = lax.broadcasted_iota(jnp.int32, (H, N), 1)
    diag = lax.broadcasted_iota(jnp.int32, (N, N), 0) == \
        lax.broadcasted_iota(jnp.int32, (N, N), 1)
    xs = []
    for layer_refs in layers:
        (eW1, eb1, eW2, eb2, aW1, ab1, aW2, ab2, lng, lnb) = [
            r[...] for r in layer_refs]
        in_d = x.shape[-1]
        out_d = eW2.shape[-1]
        w = in_d + AUX
        Wa = eW1[:in_d, :]
        Wb = eW1[in_d:, :]
        P = _dot(x, Wa - Wb) + eb1[...]
        A1 = _dot(aux, aW1[:AUX, :]) + ab1[...]
        aWb = aW1[AUX:, :]
        cat = jnp.concatenate([x, aux], axis=1)

        sq = jnp.sum(x * x, axis=1, keepdims=True)
        d2 = sq + jnp.transpose(sq) - 2.0 * _dot_t(x, x)
        d2 = jnp.where(diag, 1e10, d2)
        d2a_ref[...] = d2[:H]
        d2b_ref[...] = d2[H:]

        def body(k, carry):
            for h, ref in ((0, d2a_ref), (1, d2b_ref)):
                kc = ref[...]
                m = jnp.min(kc, axis=1, keepdims=True)
                idx = jnp.min(jnp.where(kc == m, iota_h, N), axis=1,
                              keepdims=True)
                oh = iota_h == idx
                catj = _dot(oh.astype(jnp.float32), cat)
                cj_ref[pl.ds(k * N + h * H, H), :w] = catj
                ref[...] = jnp.where(oh, jnp.float32(1e30), kc)
            return carry

        lax.fori_loop(0, K, body, 0, unroll=2)

        CJ = cj_ref[:, :w]
        z_a = jnp.zeros((in_d, 64), jnp.float32)
        z_b = jnp.zeros((AUX, out_d), jnp.float32)
        W1c = jnp.concatenate([
            jnp.concatenate([Wb, z_a], axis=1),
            jnp.concatenate([z_b, aWb], axis=1)], axis=0)
        z_c = jnp.zeros((out_d, 2 * out_d), jnp.float32)
        z_d = jnp.zeros((64, out_d), jnp.float32)
        W2c = jnp.concatenate([
            jnp.concatenate([eW2, z_c], axis=1),
            jnp.concatenate([z_d, aW2], axis=1)], axis=0)
        PA = jnp.concatenate([P, A1], axis=1)
        PAb = jnp.concatenate([PA] * K, axis=0)
        hg1 = jnp.maximum(PAb + _dot(CJ, W1c), 0.0)
        bc = jnp.concatenate([eb2, ab2], axis=1)
        hg2 = _dot(hg1, W2c) + bc
        h2 = jnp.maximum(hg2[:, :out_d], 0.0)
        gb = hg2[:, out_d:]
        mod = _sigmoid(gb[:, :out_d] + 1.0) * h2 + gb[:, out_d:]
        agg = jnp.max(mod.reshape(K, N, out_d), axis=0)
        x = jnp.maximum(_ln(agg, lng[...], lnb[...]), 0.0)
        xs.append(x)
    x_cat = jnp.concatenate(xs, axis=1)
    x_lin = jnp.maximum(_dot(x_cat, lin1_W[...]) + lin1_b[...], 0.0)
    out_ref[0, 0, :] = jnp.max(x_lin, axis=0)


def _tf_kernel(pooled_ref, pos_ref, Wq, bq, Wk, bk, Wv, bv, Wo, bo,
               ln1g, ln1b, W1, b1, W2, b2, ln2g, ln2b,
               hW0, hb0, hW1, hb1, hW2, hb2, hW3, hb3, out_ref):
    pos = pos_ref[...]
    seq = pooled_ref[...] + jnp.concatenate([pos] * B, axis=0)
    q = _dot(seq, Wq[...]) + bq[...]
    k = _dot(seq, Wk[...]) + bk[...]
    v = _dot(seq, Wv[...]) + bv[...]
    scale = 1.0 / jnp.sqrt(jnp.float32(DH))
    rows = []
    for b in range(B):
        cols = []
        for h in range(NH):
            qb = q[b * T:(b + 1) * T, h * DH:(h + 1) * DH]
            kb = k[b * T:(b + 1) * T, h * DH:(h + 1) * DH]
            vb = v[b * T:(b + 1) * T, h * DH:(h + 1) * DH]
            s = _dot_t(qb, kb) * scale
            s = s - jnp.max(s, axis=1, keepdims=True)
            e = jnp.exp(s)
            p = e / jnp.sum(e, axis=1, keepdims=True)
            cols.append(_dot(p, vb))
        rows.append(jnp.concatenate(cols, axis=1))
    ctx = jnp.concatenate(rows, axis=0)
    y = _ln(seq + _dot(ctx, Wo[...]) + bo[...], ln1g[...], ln1b[...])
    ff = _dot(jnp.maximum(_dot(y, W1[...]) + b1[...], 0.0), W2[...]) + b2[...]
    y = _ln(y + ff, ln2g[...], ln2b[...])
    feat = jnp.mean(y.reshape(B, T, DM), axis=1)
    h0 = jnp.maximum(_dot(feat, hW0[...]) + hb0[...], 0.0)
    h1 = jnp.maximum(_dot(h0, hW1[...]) + hb1[...], 0.0)
    h2 = jnp.maximum(_dot(h1, hW2[...]) + hb2[...], 0.0)
    out_ref[...] = _dot(h2, hW3[...]) + hb3[...]


def kernel(data, params):
    frames = B * T
    geom = data[..., :GEOM].reshape(frames, N, GEOM)
    aux = data[..., GEOM:GEOM + AUX].reshape(frames, N, AUX)

    edge_args = []
    edge_specs = []
    full = lambda s: pl.BlockSpec(s, lambda f: (0,) * len(s))
    for lp in params['edge']:
        for name in ('eW1', 'eb1', 'eW2', 'eb2', 'aW1', 'ab1', 'aW2', 'ab2',
                     'ln_g', 'ln_b'):
            w = lp[name]
            if w.ndim == 1:
                w = w.reshape(1, -1)
            edge_args.append(w)
            edge_specs.append(full(w.shape))

    lin1_W = params['lin1_W']
    lin1_b = params['lin1_b'].reshape(1, -1)

    pooled = pl.pallas_call(
        _edge_kernel,
        grid=(frames,),
        in_specs=[
            pl.BlockSpec((1, N, GEOM), lambda f: (f, 0, 0)),
            pl.BlockSpec((1, N, AUX), lambda f: (f, 0, 0)),
            *edge_specs,
            full(lin1_W.shape),
            full(lin1_b.shape),
        ],
        out_specs=pl.BlockSpec((1, 1, DM), lambda f: (f, 0, 0)),
        out_shape=jax.ShapeDtypeStruct((frames, 1, DM), jnp.float32),
        scratch_shapes=[pltpu.VMEM((N // 2, N), jnp.float32),
                        pltpu.VMEM((N // 2, N), jnp.float32),
                        pltpu.VMEM((K * N, CONV[0] + AUX), jnp.float32)],
    )(geom, aux, *edge_args, lin1_W, lin1_b)
    pooled = pooled.reshape(frames, DM)

    tf = params['tf']
    head = params['head']
    pos = params['pos'][0, :T, :]
    tf_args = [tf['Wq'], tf['bq'].reshape(1, -1), tf['Wk'], tf['bk'].reshape(1, -1),
               tf['Wv'], tf['bv'].reshape(1, -1), tf['Wo'], tf['bo'].reshape(1, -1),
               tf['ln1_g'].reshape(1, -1), tf['ln1_b'].reshape(1, -1),
               tf['W1'], tf['b1'].reshape(1, -1), tf['W2'], tf['b2'].reshape(1, -1),
               tf['ln2_g'].reshape(1, -1), tf['ln2_b'].reshape(1, -1),
               head[0]['W'], head[0]['b'].reshape(1, -1),
               head[1]['W'], head[1]['b'].reshape(1, -1),
               head[2]['W'], head[2]['b'].reshape(1, -1),
               head[3]['W'], head[3]['b'].reshape(1, -1)]

    out = pl.pallas_call(
        _tf_kernel,
        out_shape=jax.ShapeDtypeStruct((B, NC), jnp.float32),
    )(pooled, pos, *tf_args)
    return out


# speedup vs baseline: 1.2423x; 1.0756x over previous
import functools

import jax
import jax.numpy as jnp
from jax import lax
from jax.experimental import pallas as pl
from jax.experimental.pallas import tpu as pltpu

B, T, N, C = 4, 16, 512, 7
GEOM, AUX, K = 3, 4, 20
CONV = [32, 32, 32]
DM, NH, DH, FF, NC = 1024, 4, 256, 2048, 12


def _ln(x, g, b, eps=1e-5):
    mu = jnp.mean(x, -1, keepdims=True)
    v = jnp.mean((x - mu) ** 2, -1, keepdims=True)
    return (x - mu) / jnp.sqrt(v + eps) * g + b


def _sigmoid(x):
    return 1.0 / (1.0 + jnp.exp(-x))


def _dot(a, b):
    return jax.lax.dot_general(a, b, (((1,), (0,)), ((), ())),
                               preferred_element_type=jnp.float32)


def _dot_t(a, b):
    return jax.lax.dot_general(a, b, (((1,), (1,)), ((), ())),
                               preferred_element_type=jnp.float32)


def _edge_kernel(geom_ref, aux_ref,
                 eW1_0, eb1_0, eW2_0, eb2_0, aW1_0, ab1_0, aW2_0, ab2_0, lng_0, lnb_0,
                 eW1_1, eb1_1, eW2_1, eb2_1, aW1_1, ab1_1, aW2_1, ab2_1, lng_1, lnb_1,
                 eW1_2, eb1_2, eW2_2, eb2_2, aW1_2, ab1_2, aW2_2, ab2_2, lng_2, lnb_2,
                 lin1_W, lin1_b, out_ref, d2a_ref, d2b_ref, cj_ref):
    layers = [
        (eW1_0, eb1_0, eW2_0, eb2_0, aW1_0, ab1_0, aW2_0, ab2_0, lng_0, lnb_0),
        (eW1_1, eb1_1, eW2_1, eb2_1, aW1_1, ab1_1, aW2_1, ab2_1, lng_1, lnb_1),
        (eW1_2, eb1_2, eW2_2, eb2_2, aW1_2, ab1_2, aW2_2, ab2_2, lng_2, lnb_2),
    ]
    x = geom_ref[0]
    aux = aux_ref[0]
    H = N // 2
    iota_h = lax.broadcasted_iota(jnp.int32, (H, N), 1)
    diag = lax.broadcasted_iota(jnp.int32, (N, N), 0) == \
        lax.broadcasted_iota(jnp.int32, (N, N), 1)
    xs = []
    for layer_refs in layers:
        (eW1, eb1, eW2, eb2, aW1, ab1, aW2, ab2, lng, lnb) = [
            r[...] for r in layer_refs]
        in_d = x.shape[-1]
        out_d = eW2.shape[-1]
        w = in_d + AUX
        Wa = eW1[:in_d, :]
        Wb = eW1[in_d:, :]
        P = _dot(x, Wa - Wb) + eb1[...]
        A1 = _dot(aux, aW1[:AUX, :]) + ab1[...]
        aWb = aW1[AUX:, :]
        cat = jnp.concatenate([x, aux], axis=1)

        sq = jnp.sum(x * x, axis=1, keepdims=True)
        d2 = sq + jnp.transpose(sq) - 2.0 * _dot_t(x, x)
        d2 = jnp.where(diag, 1e10, d2)
        d2a_ref[...] = d2[:H]
        d2b_ref[...] = d2[H:]

        def body(k, carry):
            for h, ref in ((0, d2a_ref), (1, d2b_ref)):
                kc = ref[...]
                m = jnp.min(kc, axis=1, keepdims=True)
                idx = jnp.min(jnp.where(kc == m, iota_h, N), axis=1,
                              keepdims=True)
                oh = iota_h == idx
                catj = _dot(oh.astype(jnp.float32), cat)
                cj_ref[pl.ds(k * N + h * H, H), :w] = catj
                ref[...] = jnp.where(oh, jnp.float32(1e30), kc)
            return carry

        lax.fori_loop(0, K, body, 0, unroll=4)

        CJ = cj_ref[:, :w]
        z_a = jnp.zeros((in_d, 64), jnp.float32)
        z_b = jnp.zeros((AUX, out_d), jnp.float32)
        W1c = jnp.concatenate([
            jnp.concatenate([Wb, z_a], axis=1),
            jnp.concatenate([z_b, aWb], axis=1)], axis=0)
        z_c = jnp.zeros((out_d, 2 * out_d), jnp.float32)
        z_d = jnp.zeros((64, out_d), jnp.float32)
        W2c = jnp.concatenate([
            jnp.concatenate([eW2, z_c], axis=1),
            jnp.concatenate([z_d, aW2], axis=1)], axis=0)
        PA = jnp.concatenate([P, A1], axis=1)
        PAb = jnp.concatenate([PA] * K, axis=0)
        hg1 = jnp.maximum(PAb + _dot(CJ, W1c), 0.0)
        bc = jnp.concatenate([eb2, ab2], axis=1)
        hg2 = _dot(hg1, W2c) + bc
        h2 = jnp.maximum(hg2[:, :out_d], 0.0)
        gb = hg2[:, out_d:]
        mod = _sigmoid(gb[:, :out_d] + 1.0) * h2 + gb[:, out_d:]
        agg = jnp.max(mod.reshape(K, N, out_d), axis=0)
        x = jnp.maximum(_ln(agg, lng[...], lnb[...]), 0.0)
        xs.append(x)
    x_cat = jnp.concatenate(xs, axis=1)
    x_lin = jnp.maximum(_dot(x_cat, lin1_W[...]) + lin1_b[...], 0.0)
    out_ref[0, 0, :] = jnp.max(x_lin, axis=0)


def _tf_kernel(pooled_ref, pos_ref, Wq, bq, Wk, bk, Wv, bv, Wo, bo,
               ln1g, ln1b, W1, b1, W2, b2, ln2g, ln2b,
               hW0, hb0, hW1, hb1, hW2, hb2, hW3, hb3, out_ref):
    pos = pos_ref[...]
    seq = pooled_ref[...] + jnp.concatenate([pos] * B, axis=0)
    q = _dot(seq, Wq[...]) + bq[...]
    k = _dot(seq, Wk[...]) + bk[...]
    v = _dot(seq, Wv[...]) + bv[...]
    scale = 1.0 / jnp.sqrt(jnp.float32(DH))
    rows = []
    for b in range(B):
        cols = []
        for h in range(NH):
            qb = q[b * T:(b + 1) * T, h * DH:(h + 1) * DH]
            kb = k[b * T:(b + 1) * T, h * DH:(h + 1) * DH]
            vb = v[b * T:(b + 1) * T, h * DH:(h + 1) * DH]
            s = _dot_t(qb, kb) * scale
            s = s - jnp.max(s, axis=1, keepdims=True)
            e = jnp.exp(s)
            p = e / jnp.sum(e, axis=1, keepdims=True)
            cols.append(_dot(p, vb))
        rows.append(jnp.concatenate(cols, axis=1))
    ctx = jnp.concatenate(rows, axis=0)
    y = _ln(seq + _dot(ctx, Wo[...]) + bo[...], ln1g[...], ln1b[...])
    ff = _dot(jnp.maximum(_dot(y, W1[...]) + b1[...], 0.0), W2[...]) + b2[...]
    y = _ln(y + ff, ln2g[...], ln2b[...])
    feat = jnp.mean(y.reshape(B, T, DM), axis=1)
    h0 = jnp.maximum(_dot(feat, hW0[...]) + hb0[...], 0.0)
    h1 = jnp.maximum(_dot(h0, hW1[...]) + hb1[...], 0.0)
    h2 = jnp.maximum(_dot(h1, hW2[...]) + hb2[...], 0.0)
    out_ref[...] = _dot(h2, hW3[...]) + hb3[...]


def kernel(data, params):
    frames = B * T
    geom = data[..., :GEOM].reshape(frames, N, GEOM)
    aux = data[..., GEOM:GEOM + AUX].reshape(frames, N, AUX)

    edge_args = []
    edge_specs = []
    full = lambda s: pl.BlockSpec(s, lambda f: (0,) * len(s))
    for lp in params['edge']:
        for name in ('eW1', 'eb1', 'eW2', 'eb2', 'aW1', 'ab1', 'aW2', 'ab2',
                     'ln_g', 'ln_b'):
            w = lp[name]
            if w.ndim == 1:
                w = w.reshape(1, -1)
            edge_args.append(w)
            edge_specs.append(full(w.shape))

    lin1_W = params['lin1_W']
    lin1_b = params['lin1_b'].reshape(1, -1)

    pooled = pl.pallas_call(
        _edge_kernel,
        grid=(frames,),
        in_specs=[
            pl.BlockSpec((1, N, GEOM), lambda f: (f, 0, 0)),
            pl.BlockSpec((1, N, AUX), lambda f: (f, 0, 0)),
            *edge_specs,
            full(lin1_W.shape),
            full(lin1_b.shape),
        ],
        out_specs=pl.BlockSpec((1, 1, DM), lambda f: (f, 0, 0)),
        out_shape=jax.ShapeDtypeStruct((frames, 1, DM), jnp.float32),
        scratch_shapes=[pltpu.VMEM((N // 2, N), jnp.float32),
                        pltpu.VMEM((N // 2, N), jnp.float32),
                        pltpu.VMEM((K * N, CONV[0] + AUX), jnp.float32)],
    )(geom, aux, *edge_args, lin1_W, lin1_b)
    pooled = pooled.reshape(frames, DM)

    tf = params['tf']
    head = params['head']
    pos = params['pos'][0, :T, :]
    tf_args = [tf['Wq'], tf['bq'].reshape(1, -1), tf['Wk'], tf['bk'].reshape(1, -1),
               tf['Wv'], tf['bv'].reshape(1, -1), tf['Wo'], tf['bo'].reshape(1, -1),
               tf['ln1_g'].reshape(1, -1), tf['ln1_b'].reshape(1, -1),
               tf['W1'], tf['b1'].reshape(1, -1), tf['W2'], tf['b2'].reshape(1, -1),
               tf['ln2_g'].reshape(1, -1), tf['ln2_b'].reshape(1, -1),
               head[0]['W'], head[0]['b'].reshape(1, -1),
               head[1]['W'], head[1]['b'].reshape(1, -1),
               head[2]['W'], head[2]['b'].reshape(1, -1),
               head[3]['W'], head[3]['b'].reshape(1, -1)]

    out = pl.pallas_call(
        _tf_kernel,
        out_shape=jax.ShapeDtypeStruct((B, NC), jnp.float32),
    )(pooled, pos, *tf_args)
    return out


# speedup vs baseline: 1.3333x; 1.0732x over previous
import functools

import jax
import jax.numpy as jnp
from jax import lax
from jax.experimental import pallas as pl
from jax.experimental.pallas import tpu as pltpu

B, T, N, C = 4, 16, 512, 7
GEOM, AUX, K = 3, 4, 20
CONV = [32, 32, 32]
DM, NH, DH, FF, NC = 1024, 4, 256, 2048, 12


def _ln(x, g, b, eps=1e-5):
    mu = jnp.mean(x, -1, keepdims=True)
    v = jnp.mean((x - mu) ** 2, -1, keepdims=True)
    return (x - mu) / jnp.sqrt(v + eps) * g + b


def _sigmoid(x):
    return 1.0 / (1.0 + jnp.exp(-x))


def _dot(a, b):
    return jax.lax.dot_general(a, b, (((1,), (0,)), ((), ())),
                               preferred_element_type=jnp.float32)


def _dot_t(a, b):
    return jax.lax.dot_general(a, b, (((1,), (1,)), ((), ())),
                               preferred_element_type=jnp.float32)


def _edge_kernel(geom_ref, aux_ref,
                 eW1_0, eb1_0, eW2_0, eb2_0, aW1_0, ab1_0, aW2_0, ab2_0, lng_0, lnb_0,
                 eW1_1, eb1_1, eW2_1, eb2_1, aW1_1, ab1_1, aW2_1, ab2_1, lng_1, lnb_1,
                 eW1_2, eb1_2, eW2_2, eb2_2, aW1_2, ab1_2, aW2_2, ab2_2, lng_2, lnb_2,
                 lin1_W, lin1_b, out_ref, d2a_ref, d2b_ref, cj_ref):
    layers = [
        (eW1_0, eb1_0, eW2_0, eb2_0, aW1_0, ab1_0, aW2_0, ab2_0, lng_0, lnb_0),
        (eW1_1, eb1_1, eW2_1, eb2_1, aW1_1, ab1_1, aW2_1, ab2_1, lng_1, lnb_1),
        (eW1_2, eb1_2, eW2_2, eb2_2, aW1_2, ab1_2, aW2_2, ab2_2, lng_2, lnb_2),
    ]
    x = geom_ref[0]
    aux = aux_ref[0]
    H = N // 2
    iota_h = lax.broadcasted_iota(jnp.int32, (H, N), 1)
    diag = lax.broadcasted_iota(jnp.int32, (N, N), 0) == \
        lax.broadcasted_iota(jnp.int32, (N, N), 1)
    xs = []
    for layer_refs in layers:
        (eW1, eb1, eW2, eb2, aW1, ab1, aW2, ab2, lng, lnb) = [
            r[...] for r in layer_refs]
        in_d = x.shape[-1]
        out_d = eW2.shape[-1]
        w = in_d + AUX
        Wa = eW1[:in_d, :]
        Wb = eW1[in_d:, :]
        P = _dot(x, Wa - Wb) + eb1[...]
        A1 = _dot(aux, aW1[:AUX, :]) + ab1[...]
        aWb = aW1[AUX:, :]
        cat = jnp.concatenate([x, aux], axis=1)

        sq = jnp.sum(x * x, axis=1, keepdims=True)
        d2 = sq + jnp.transpose(sq) - 2.0 * _dot_t(x, x)
        d2 = jnp.where(diag, 1e10, d2)
        d2a_ref[...] = d2[:H]
        d2b_ref[...] = d2[H:]

        def body(k, carry):
            for h, ref in ((0, d2a_ref), (1, d2b_ref)):
                kc = ref[...]
                m = jnp.min(kc, axis=1, keepdims=True)
                idx = jnp.min(jnp.where(kc == m, iota_h, N), axis=1,
                              keepdims=True)
                oh = iota_h == idx
                catj = _dot(oh.astype(jnp.float32), cat)
                cj_ref[pl.ds(k * N + h * H, H), :w] = catj
                ref[...] = jnp.where(oh, jnp.float32(1e30), kc)
            return carry

        lax.fori_loop(0, K, body, 0, unroll=10)

        CJ = cj_ref[:, :w]
        z_a = jnp.zeros((in_d, 64), jnp.float32)
        z_b = jnp.zeros((AUX, out_d), jnp.float32)
        W1c = jnp.concatenate([
            jnp.concatenate([Wb, z_a], axis=1),
            jnp.concatenate([z_b, aWb], axis=1)], axis=0)
        z_c = jnp.zeros((out_d, 2 * out_d), jnp.float32)
        z_d = jnp.zeros((64, out_d), jnp.float32)
        W2c = jnp.concatenate([
            jnp.concatenate([eW2, z_c], axis=1),
            jnp.concatenate([z_d, aW2], axis=1)], axis=0)
        PA = jnp.concatenate([P, A1], axis=1)
        PAb = jnp.concatenate([PA] * K, axis=0)
        hg1 = jnp.maximum(PAb + _dot(CJ, W1c), 0.0)
        bc = jnp.concatenate([eb2, ab2], axis=1)
        hg2 = _dot(hg1, W2c) + bc
        h2 = jnp.maximum(hg2[:, :out_d], 0.0)
        gb = hg2[:, out_d:]
        mod = _sigmoid(gb[:, :out_d] + 1.0) * h2 + gb[:, out_d:]
        agg = jnp.max(mod.reshape(K, N, out_d), axis=0)
        x = jnp.maximum(_ln(agg, lng[...], lnb[...]), 0.0)
        xs.append(x)
    x_cat = jnp.concatenate(xs, axis=1)
    x_lin = jnp.maximum(_dot(x_cat, lin1_W[...]) + lin1_b[...], 0.0)
    out_ref[0, 0, :] = jnp.max(x_lin, axis=0)


def _tf_kernel(pooled_ref, pos_ref, Wq, bq, Wk, bk, Wv, bv, Wo, bo,
               ln1g, ln1b, W1, b1, W2, b2, ln2g, ln2b,
               hW0, hb0, hW1, hb1, hW2, hb2, hW3, hb3, out_ref):
    pos = pos_ref[...]
    seq = pooled_ref[...] + jnp.concatenate([pos] * B, axis=0)
    q = _dot(seq, Wq[...]) + bq[...]
    k = _dot(seq, Wk[...]) + bk[...]
    v = _dot(seq, Wv[...]) + bv[...]
    scale = 1.0 / jnp.sqrt(jnp.float32(DH))
    rows = []
    for b in range(B):
        cols = []
        for h in range(NH):
            qb = q[b * T:(b + 1) * T, h * DH:(h + 1) * DH]
            kb = k[b * T:(b + 1) * T, h * DH:(h + 1) * DH]
            vb = v[b * T:(b + 1) * T, h * DH:(h + 1) * DH]
            s = _dot_t(qb, kb) * scale
            s = s - jnp.max(s, axis=1, keepdims=True)
            e = jnp.exp(s)
            p = e / jnp.sum(e, axis=1, keepdims=True)
            cols.append(_dot(p, vb))
        rows.append(jnp.concatenate(cols, axis=1))
    ctx = jnp.concatenate(rows, axis=0)
    y = _ln(seq + _dot(ctx, Wo[...]) + bo[...], ln1g[...], ln1b[...])
    ff = _dot(jnp.maximum(_dot(y, W1[...]) + b1[...], 0.0), W2[...]) + b2[...]
    y = _ln(y + ff, ln2g[...], ln2b[...])
    feat = jnp.mean(y.reshape(B, T, DM), axis=1)
    h0 = jnp.maximum(_dot(feat, hW0[...]) + hb0[...], 0.0)
    h1 = jnp.maximum(_dot(h0, hW1[...]) + hb1[...], 0.0)
    h2 = jnp.maximum(_dot(h1, hW2[...]) + hb2[...], 0.0)
    out_ref[...] = _dot(h2, hW3[...]) + hb3[...]


def kernel(data, params):
    frames = B * T
    geom = data[..., :GEOM].reshape(frames, N, GEOM)
    aux = data[..., GEOM:GEOM + AUX].reshape(frames, N, AUX)

    edge_args = []
    edge_specs = []
    full = lambda s: pl.BlockSpec(s, lambda f: (0,) * len(s))
    for lp in params['edge']:
        for name in ('eW1', 'eb1', 'eW2', 'eb2', 'aW1', 'ab1', 'aW2', 'ab2',
                     'ln_g', 'ln_b'):
            w = lp[name]
            if w.ndim == 1:
                w = w.reshape(1, -1)
            edge_args.append(w)
            edge_specs.append(full(w.shape))

    lin1_W = params['lin1_W']
    lin1_b = params['lin1_b'].reshape(1, -1)

    pooled = pl.pallas_call(
        _edge_kernel,
        grid=(frames,),
        in_specs=[
            pl.BlockSpec((1, N, GEOM), lambda f: (f, 0, 0)),
            pl.BlockSpec((1, N, AUX), lambda f: (f, 0, 0)),
            *edge_specs,
            full(lin1_W.shape),
            full(lin1_b.shape),
        ],
        out_specs=pl.BlockSpec((1, 1, DM), lambda f: (f, 0, 0)),
        out_shape=jax.ShapeDtypeStruct((frames, 1, DM), jnp.float32),
        scratch_shapes=[pltpu.VMEM((N // 2, N), jnp.float32),
                        pltpu.VMEM((N // 2, N), jnp.float32),
                        pltpu.VMEM((K * N, CONV[0] + AUX), jnp.float32)],
    )(geom, aux, *edge_args, lin1_W, lin1_b)
    pooled = pooled.reshape(frames, DM)

    tf = params['tf']
    head = params['head']
    pos = params['pos'][0, :T, :]
    tf_args = [tf['Wq'], tf['bq'].reshape(1, -1), tf['Wk'], tf['bk'].reshape(1, -1),
               tf['Wv'], tf['bv'].reshape(1, -1), tf['Wo'], tf['bo'].reshape(1, -1),
               tf['ln1_g'].reshape(1, -1), tf['ln1_b'].reshape(1, -1),
               tf['W1'], tf['b1'].reshape(1, -1), tf['W2'], tf['b2'].reshape(1, -1),
               tf['ln2_g'].reshape(1, -1), tf['ln2_b'].reshape(1, -1),
               head[0]['W'], head[0]['b'].reshape(1, -1),
               head[1]['W'], head[1]['b'].reshape(1, -1),
               head[2]['W'], head[2]['b'].reshape(1, -1),
               head[3]['W'], head[3]['b'].reshape(1, -1)]

    out = pl.pallas_call(
        _tf_kernel,
        out_shape=jax.ShapeDtypeStruct((B, NC), jnp.float32),
    )(pooled, pos, *tf_args)
    return out
